# baseline (speedup 1.0000x reference)
import functools

import jax
import jax.numpy as jnp
from jax import lax
from jax.experimental import pallas as pl
from jax.experimental.pallas import tpu as pltpu
from jax.experimental.pallas import tpu_sc as plsc

_VOCAB = 100000
_D = 64
_K = _D + 2
_B = 4096
_CTX = 20
_VB = 512
_VPAD = ((_VOCAB + _VB - 1) // _VB) * _VB
_NSTEPS = _VPAD // _VB
_OB = 128
_OV = 25088
_LOG2E = 1.4426950408889634


@functools.lru_cache(maxsize=1)
def _make_gather_sum():
    info = plsc.get_sparse_core_info()
    nc, ns, L = info.num_cores, info.num_subcores, info.num_lanes
    nw = nc * ns
    b_per_w = _B // nw
    rows_per_w = b_per_w * _CTX
    G = 128
    ng = rows_per_w // G
    nhalf = 2
    ng_h = ng // nhalf
    rows_h = rows_per_w // nhalf
    b_h = b_per_w // nhalf

    mesh = plsc.VectorSubcoreMesh(core_axis_name="c", subcore_axis_name="s")

    @functools.partial(
        pl.kernel,
        mesh=mesh,
        out_type=jax.ShapeDtypeStruct((_B, _D), jnp.float32),
        scratch_types=[
            pltpu.VMEM((ng, G), jnp.int32),
            pltpu.VMEM((rows_h, _D), jnp.float32),
            pltpu.VMEM((b_per_w, _D), jnp.float32),
            pltpu.SemaphoreType.DMA,
        ],
        compiler_params=pltpu.CompilerParams(use_tc_tiling_on_sc=False),
    )
    def gather_sum(idx_hbm, table_hbm, out_hbm, idx_v, buf_v, acc_v, sem):
        wid = lax.axis_index("s") * nc + lax.axis_index("c")
        pltpu.sync_copy(idx_hbm.at[wid], idx_v)
        for h in range(nhalf):
            copies = []
            for j in range(ng_h):
                copies.append(
                    pltpu.async_copy(
                        table_hbm.at[idx_v.at[h * ng_h + j]],
                        buf_v.at[pl.ds(j * G, G)],
                        sem,
                    )
                )
            for c in copies:
                c.wait()

            def body(b, carry, h=h):
                r0 = b * _CTX
                for l in range(_D // L):
                    sl = pl.ds(l * L, L)
                    a = buf_v[r0, sl]
                    for t in range(1, _CTX):
                        a = a + buf_v[r0 + t, sl]
                    acc_v[h * b_h + b, sl] = a
                return carry

            lax.fori_loop(0, b_h, body, 0)

        pltpu.sync_copy(acc_v, out_hbm.at[pl.ds(wid * b_per_w, b_per_w)])

    return gather_sum


_NB = _B // _OB
_NV = _VPAD // _OV


def _fold(p):
    acc = p[:, 0:128]
    for c in range(128, _OV, 128):
        acc = acc + p[:, c:c + 128]
    return acc


def _lse0_body(xs_ref, w_ref, o_ref, s_ref):
    v = pl.program_id(0)
    d = lax.dot_general(
        xs_ref[...], w_ref[...], (((1,), (1,)), ((), ())),
        preferred_element_type=jnp.float32,
    )
    part = _fold(jnp.exp2(d))
    s_ref[...] = jnp.where(v == 0, part, s_ref[...] + part)

    @pl.when(v == _NV - 1)
    def _():
        o_ref[...] = jnp.log(jnp.sum(s_ref[...], axis=1, keepdims=True))


def _lse0_pass(xs, waug):
    return pl.pallas_call(
        _lse0_body,
        grid=(_NV,),
        in_specs=[
            pl.BlockSpec((_OB, _K), lambda v: (0, 0)),
            pl.BlockSpec((_OV, _K), lambda v: (v, 0)),
        ],
        out_specs=pl.BlockSpec((_OB, 1), lambda v: (0, 0)),
        out_shape=jax.ShapeDtypeStruct((_OB, 1), jnp.float32),
        scratch_shapes=[
            pltpu.VMEM((_OB, 128), jnp.float32),
        ],
        compiler_params=pltpu.CompilerParams(
            dimension_semantics=("arbitrary",),
        ),
    )(xs, waug)


def _out_body(x_ref, xs_ref, w_ref, lse0_ref, o_ref, s_ref, lse_ref, d_ref):
    b = pl.program_id(0)
    v = pl.program_id(1)
    t = b * _NV + v

    @pl.when(t > 0)
    def _():
        part = _fold(jnp.exp2(d_ref[...]))

        @pl.when(v == 0)
        def _():
            lse_ref[...] = jnp.log(
                jnp.sum(s_ref[...] + part, axis=1, keepdims=True))

        @pl.when(v > 0)
        def _():
            s_ref[...] = jnp.where(v == 1, part, s_ref[...] + part)

    logits = lax.dot_general(
        x_ref[...], w_ref[...], (((1,), (1,)), ((), ())),
        preferred_element_type=jnp.float32,
    )
    lse = jnp.where(b == 0, lse0_ref[...], lse_ref[...])
    o_ref[...] = (logits - lse).astype(jnp.bfloat16)

    d_ref[...] = lax.dot_general(
        xs_ref[...], w_ref[...], (((1,), (1,)), ((), ())),
        preferred_element_type=jnp.float32,
    )


def _out_pass(xaug, xs, waug, lse0):
    return pl.pallas_call(
        _out_body,
        grid=(_NB, _NV),
        in_specs=[
            pl.BlockSpec((_OB, _K), lambda b, v: (b, 0)),
            pl.BlockSpec((_OB, _K), lambda b, v: (jnp.minimum(b + 1, _NB - 1), 0)),
            pl.BlockSpec((_OV, _K), lambda b, v: (v, 0)),
            pl.BlockSpec((_OB, 1), lambda b, v: (0, 0)),
        ],
        out_specs=pl.BlockSpec((_OB, _OV), lambda b, v: (b, v)),
        out_shape=jax.ShapeDtypeStruct((_B, _VOCAB), jnp.bfloat16),
        scratch_shapes=[
            pltpu.VMEM((_OB, 128), jnp.float32),
            pltpu.VMEM((_OB, 1), jnp.float32),
            pltpu.VMEM((_OB, _OV), jnp.float32),
        ],
        compiler_params=pltpu.CompilerParams(
            dimension_semantics=("arbitrary", "arbitrary"),
        ),
    )(xaug, xs, waug, lse0)


def kernel(word_indices, emb_table, lin_w, lin_b):
    idx2d = word_indices.astype(jnp.int32).reshape(32, -1, 128)
    sum_emb = _make_gather_sum()(idx2d, emb_table)

    w_pad = jnp.pad(lin_w, ((0, _VPAD - _VOCAB), (0, 0)))
    b_pad = jnp.pad(lin_b, (0, _VPAD - _VOCAB), constant_values=-1e30)
    waug = jnp.concatenate(
        [w_pad, b_pad[:, None], jnp.full((_VPAD, 1), -1.0, jnp.float32)],
        axis=1,
    ).astype(jnp.bfloat16)

    ones = jnp.ones((_B, 1), jnp.float32)
    xs = jnp.concatenate(
        [sum_emb * _LOG2E, ones * _LOG2E, ones * 0.0], axis=1
    ).astype(jnp.bfloat16)
    xaug = jnp.concatenate([sum_emb, ones, ones * 0.0], axis=1).astype(
        jnp.bfloat16)

    lse0 = _lse0_pass(xs[: _OB], waug)
    return _out_pass(xaug, xs, waug, lse0).astype(jnp.float32)


# speedup vs baseline: 1.1002x; 1.1002x over previous
import functools

import jax
import jax.numpy as jnp
from jax import lax
from jax.experimental import pallas as pl
from jax.experimental.pallas import tpu as pltpu
from jax.experimental.pallas import tpu_sc as plsc

_VOCAB = 100000
_D = 64
_K = _D + 2
_B = 4096
_CTX = 20
_VB = 512
_VPAD = ((_VOCAB + _VB - 1) // _VB) * _VB
_NSTEPS = _VPAD // _VB
_OB = 128
_OV = 25088
_LOG2E = 1.4426950408889634


@functools.lru_cache(maxsize=1)
def _make_gather_sum():
    info = plsc.get_sparse_core_info()
    nc, ns, L = info.num_cores, info.num_subcores, info.num_lanes
    nw = nc * ns
    b_per_w = _B // nw
    rows_per_w = b_per_w * _CTX
    G = 128
    ng = rows_per_w // G
    nhalf = 2
    ng_h = ng // nhalf
    rows_h = rows_per_w // nhalf
    b_h = b_per_w // nhalf

    mesh = plsc.VectorSubcoreMesh(core_axis_name="c", subcore_axis_name="s")

    @functools.partial(
        pl.kernel,
        mesh=mesh,
        out_type=jax.ShapeDtypeStruct((_B, _D), jnp.float32),
        scratch_types=[
            pltpu.VMEM((ng, G), jnp.int32),
            pltpu.VMEM((rows_h, _D), jnp.float32),
            pltpu.VMEM((b_per_w, _D), jnp.float32),
            pltpu.SemaphoreType.DMA,
        ],
        compiler_params=pltpu.CompilerParams(use_tc_tiling_on_sc=False),
    )
    def gather_sum(idx_hbm, table_hbm, out_hbm, idx_v, buf_v, acc_v, sem):
        wid = lax.axis_index("s") * nc + lax.axis_index("c")
        pltpu.sync_copy(idx_hbm.at[wid], idx_v)
        for h in range(nhalf):
            copies = []
            for j in range(ng_h):
                copies.append(
                    pltpu.async_copy(
                        table_hbm.at[idx_v.at[h * ng_h + j]],
                        buf_v.at[pl.ds(j * G, G)],
                        sem,
                    )
                )
            for c in copies:
                c.wait()

            def body(b, carry, h=h):
                r0 = b * _CTX
                for l in range(_D // L):
                    sl = pl.ds(l * L, L)
                    a = buf_v[r0, sl]
                    for t in range(1, _CTX):
                        a = a + buf_v[r0 + t, sl]
                    acc_v[h * b_h + b, sl] = a
                return carry

            lax.fori_loop(0, b_h, body, 0)

        pltpu.sync_copy(acc_v, out_hbm.at[pl.ds(wid * b_per_w, b_per_w)])

    return gather_sum


_NB = _B // _OB
_NV = _VPAD // _OV


def _lse_body(x_ref, w_ref, o_ref, s_ref):
    j = pl.program_id(0)
    d = lax.dot_general(
        x_ref[...], w_ref[...], (((1,), (1,)), ((), ())),
        preferred_element_type=jnp.float32,
    )
    p = jnp.exp2(d)
    part = (p[:, 0:128] + p[:, 128:256]) + (p[:, 256:384] + p[:, 384:512])
    s_ref[...] = jnp.where(j == 0, part, s_ref[...] + part)

    @pl.when(j == _NSTEPS - 1)
    def _():
        o_ref[...] = jnp.log(jnp.sum(s_ref[...], axis=1, keepdims=True))


def _lse_pass(xs, waug):
    return pl.pallas_call(
        _lse_body,
        grid=(_NSTEPS,),
        in_specs=[
            pl.BlockSpec((_B, _K), lambda j: (0, 0)),
            pl.BlockSpec((_VB, _K), lambda j: (j, 0)),
        ],
        out_specs=pl.BlockSpec((_B, 1), lambda j: (0, 0)),
        out_shape=jax.ShapeDtypeStruct((_B, 1), jnp.float32),
        scratch_shapes=[
            pltpu.VMEM((_B, 128), jnp.float32),
        ],
        compiler_params=pltpu.CompilerParams(
            dimension_semantics=("arbitrary",),
        ),
    )(xs, waug)


def _out_body(x_ref, w_ref, o_ref):
    o_ref[...] = lax.dot_general(
        x_ref[...], w_ref[...], (((1,), (1,)), ((), ())),
        preferred_element_type=jnp.float32,
    ).astype(jnp.bfloat16)


def _out_pass(xaug, waug):
    return pl.pallas_call(
        _out_body,
        grid=(_NV, _NB),
        in_specs=[
            pl.BlockSpec((_OB, _K), lambda v, b: (b, 0)),
            pl.BlockSpec((_OV, _K), lambda v, b: (v, 0)),
        ],
        out_specs=pl.BlockSpec((_OB, _OV), lambda v, b: (b, v)),
        out_shape=jax.ShapeDtypeStruct((_B, _VOCAB), jnp.bfloat16),
        compiler_params=pltpu.CompilerParams(
            dimension_semantics=("arbitrary", "arbitrary"),
        ),
    )(xaug, waug)


def kernel(word_indices, emb_table, lin_w, lin_b):
    idx2d = word_indices.astype(jnp.int32).reshape(32, -1, 128)
    sum_emb = _make_gather_sum()(idx2d, emb_table)

    w_pad = jnp.pad(lin_w, ((0, _VPAD - _VOCAB), (0, 0)))
    b_pad = jnp.pad(lin_b, (0, _VPAD - _VOCAB), constant_values=-1e30)
    waug = jnp.concatenate(
        [w_pad, b_pad[:, None], jnp.full((_VPAD, 1), -1.0, jnp.float32)],
        axis=1,
    ).astype(jnp.bfloat16)

    ones = jnp.ones((_B, 1), jnp.float32)
    xs = jnp.concatenate(
        [sum_emb * _LOG2E, ones * _LOG2E, ones * 0.0], axis=1
    ).astype(jnp.bfloat16)
    lse = _lse_pass(xs, waug)
    xaug = jnp.concatenate([sum_emb, ones, lse], axis=1).astype(jnp.bfloat16)
    return _out_pass(xaug, waug).astype(jnp.float32)


# speedup vs baseline: 1.1040x; 1.0034x over previous
import functools

import jax
import jax.numpy as jnp
from jax import lax
from jax.experimental import pallas as pl
from jax.experimental.pallas import tpu as pltpu
from jax.experimental.pallas import tpu_sc as plsc

_VOCAB = 100000
_D = 64
_K = _D + 2
_B = 4096
_CTX = 20
_VB = 1024
_VPAD = ((_VOCAB + _VB - 1) // _VB) * _VB
_NSTEPS = _VPAD // _VB
_OB = 128
_OV = 25088
_LOG2E = 1.4426950408889634


@functools.lru_cache(maxsize=1)
def _make_gather_sum():
    info = plsc.get_sparse_core_info()
    nc, ns, L = info.num_cores, info.num_subcores, info.num_lanes
    nw = nc * ns
    b_per_w = _B // nw
    rows_per_w = b_per_w * _CTX
    G = 128
    ng = rows_per_w // G
    nhalf = 2
    ng_h = ng // nhalf
    rows_h = rows_per_w // nhalf
    b_h = b_per_w // nhalf

    mesh = plsc.VectorSubcoreMesh(core_axis_name="c", subcore_axis_name="s")

    @functools.partial(
        pl.kernel,
        mesh=mesh,
        out_type=jax.ShapeDtypeStruct((_B, _D), jnp.float32),
        scratch_types=[
            pltpu.VMEM((ng, G), jnp.int32),
            pltpu.VMEM((rows_h, _D), jnp.float32),
            pltpu.VMEM((b_per_w, _D), jnp.float32),
            pltpu.SemaphoreType.DMA,
        ],
        compiler_params=pltpu.CompilerParams(use_tc_tiling_on_sc=False),
    )
    def gather_sum(idx_hbm, table_hbm, out_hbm, idx_v, buf_v, acc_v, sem):
        wid = lax.axis_index("s") * nc + lax.axis_index("c")
        pltpu.sync_copy(idx_hbm.at[wid], idx_v)
        for h in range(nhalf):
            copies = []
            for j in range(ng_h):
                copies.append(
                    pltpu.async_copy(
                        table_hbm.at[idx_v.at[h * ng_h + j]],
                        buf_v.at[pl.ds(j * G, G)],
                        sem,
                    )
                )
            for c in copies:
                c.wait()

            def body(b, carry, h=h):
                r0 = b * _CTX
                for l in range(_D // L):
                    sl = pl.ds(l * L, L)
                    a = buf_v[r0, sl]
                    for t in range(1, _CTX):
                        a = a + buf_v[r0 + t, sl]
                    acc_v[h * b_h + b, sl] = a
                return carry

            lax.fori_loop(0, b_h, body, 0)

        pltpu.sync_copy(acc_v, out_hbm.at[pl.ds(wid * b_per_w, b_per_w)])

    return gather_sum


_NB = _B // _OB
_NV = _VPAD // _OV


def _lse_body(x_ref, w_ref, o_ref, s_ref):
    j = pl.program_id(0)
    d = lax.dot_general(
        x_ref[...], w_ref[...], (((1,), (1,)), ((), ())),
        preferred_element_type=jnp.float32,
    )
    p = jnp.exp2(d)
    cols = [p[:, c:c + 128] for c in range(0, _VB, 128)]
    while len(cols) > 1:
        cols = [a + b for a, b in zip(cols[::2], cols[1::2])]
    s_ref[...] = jnp.where(j == 0, cols[0], s_ref[...] + cols[0])

    @pl.when(j == _NSTEPS - 1)
    def _():
        o_ref[...] = jnp.log(jnp.sum(s_ref[...], axis=1, keepdims=True))


def _lse_pass(xs, waug):
    return pl.pallas_call(
        _lse_body,
        grid=(_NSTEPS,),
        in_specs=[
            pl.BlockSpec((_B, _K), lambda j: (0, 0)),
            pl.BlockSpec((_VB, _K), lambda j: (j, 0)),
        ],
        out_specs=pl.BlockSpec((_B, 1), lambda j: (0, 0)),
        out_shape=jax.ShapeDtypeStruct((_B, 1), jnp.float32),
        scratch_shapes=[
            pltpu.VMEM((_B, 128), jnp.float32),
        ],
        compiler_params=pltpu.CompilerParams(
            dimension_semantics=("arbitrary",),
        ),
    )(xs, waug)


def _out_body(x_ref, w_ref, o_ref):
    o_ref[...] = lax.dot_general(
        x_ref[...], w_ref[...], (((1,), (1,)), ((), ())),
        preferred_element_type=jnp.float32,
    ).astype(jnp.bfloat16)


def _out_pass(xaug, waug):
    return pl.pallas_call(
        _out_body,
        grid=(_NV, _NB),
        in_specs=[
            pl.BlockSpec((_OB, _K), lambda v, b: (b, 0)),
            pl.BlockSpec((_OV, _K), lambda v, b: (v, 0)),
        ],
        out_specs=pl.BlockSpec((_OB, _OV), lambda v, b: (b, v)),
        out_shape=jax.ShapeDtypeStruct((_B, _VOCAB), jnp.bfloat16),
        compiler_params=pltpu.CompilerParams(
            dimension_semantics=("arbitrary", "arbitrary"),
        ),
    )(xaug, waug)


def kernel(word_indices, emb_table, lin_w, lin_b):
    idx2d = word_indices.astype(jnp.int32).reshape(32, -1, 128)
    sum_emb = _make_gather_sum()(idx2d, emb_table)

    w_pad = jnp.pad(lin_w, ((0, _VPAD - _VOCAB), (0, 0)))
    b_pad = jnp.pad(lin_b, (0, _VPAD - _VOCAB), constant_values=-1e30)
    waug = jnp.concatenate(
        [w_pad, b_pad[:, None], jnp.full((_VPAD, 1), -1.0, jnp.float32)],
        axis=1,
    ).astype(jnp.bfloat16)

    ones = jnp.ones((_B, 1), jnp.float32)
    xs = jnp.concatenate(
        [sum_emb * _LOG2E, ones * _LOG2E, ones * 0.0], axis=1
    ).astype(jnp.bfloat16)
    lse = _lse_pass(xs, waug)
    xaug = jnp.concatenate([sum_emb, ones, lse], axis=1).astype(jnp.bfloat16)
    return _out_pass(xaug, waug).astype(jnp.float32)


# speedup vs baseline: 1.1105x; 1.0059x over previous
import functools

import jax
import jax.numpy as jnp
from jax import lax
from jax.experimental import pallas as pl
from jax.experimental.pallas import tpu as pltpu
from jax.experimental.pallas import tpu_sc as plsc

_VOCAB = 100000
_D = 64
_K = _D + 2
_B = 4096
_CTX = 20
_VB = 2048
_VPAD = ((_VOCAB + _VB - 1) // _VB) * _VB
_NSTEPS = _VPAD // _VB
_OB = 128
_OV = 25088
_LOG2E = 1.4426950408889634


@functools.lru_cache(maxsize=1)
def _make_gather_sum():
    info = plsc.get_sparse_core_info()
    nc, ns, L = info.num_cores, info.num_subcores, info.num_lanes
    nw = nc * ns
    b_per_w = _B // nw
    rows_per_w = b_per_w * _CTX
    G = 128
    ng = rows_per_w // G
    nhalf = 2
    ng_h = ng // nhalf
    rows_h = rows_per_w // nhalf
    b_h = b_per_w // nhalf

    mesh = plsc.VectorSubcoreMesh(core_axis_name="c", subcore_axis_name="s")

    @functools.partial(
        pl.kernel,
        mesh=mesh,
        out_type=jax.ShapeDtypeStruct((_B, _D), jnp.float32),
        scratch_types=[
            pltpu.VMEM((ng, G), jnp.int32),
            pltpu.VMEM((rows_h, _D), jnp.float32),
            pltpu.VMEM((b_per_w, _D), jnp.float32),
            pltpu.SemaphoreType.DMA,
        ],
        compiler_params=pltpu.CompilerParams(use_tc_tiling_on_sc=False),
    )
    def gather_sum(idx_hbm, table_hbm, out_hbm, idx_v, buf_v, acc_v, sem):
        wid = lax.axis_index("s") * nc + lax.axis_index("c")
        pltpu.sync_copy(idx_hbm.at[wid], idx_v)
        for h in range(nhalf):
            copies = []
            for j in range(ng_h):
                copies.append(
                    pltpu.async_copy(
                        table_hbm.at[idx_v.at[h * ng_h + j]],
                        buf_v.at[pl.ds(j * G, G)],
                        sem,
                    )
                )
            for c in copies:
                c.wait()

            def body(b, carry, h=h):
                r0 = b * _CTX
                for l in range(_D // L):
                    sl = pl.ds(l * L, L)
                    a = buf_v[r0, sl]
                    for t in range(1, _CTX):
                        a = a + buf_v[r0 + t, sl]
                    acc_v[h * b_h + b, sl] = a
                return carry

            lax.fori_loop(0, b_h, body, 0)

        pltpu.sync_copy(acc_v, out_hbm.at[pl.ds(wid * b_per_w, b_per_w)])

    return gather_sum


_NB = _B // _OB
_NV = _VPAD // _OV


def _lse_body(x_ref, w_ref, o_ref, s_ref):
    j = pl.program_id(0)
    d = lax.dot_general(
        x_ref[...], w_ref[...], (((1,), (1,)), ((), ())),
        preferred_element_type=jnp.float32,
    )
    p = jnp.exp2(d)
    cols = [p[:, c:c + 128] for c in range(0, _VB, 128)]
    while len(cols) > 1:
        cols = [a + b for a, b in zip(cols[::2], cols[1::2])]
    s_ref[...] = jnp.where(j == 0, cols[0], s_ref[...] + cols[0])

    @pl.when(j == _NSTEPS - 1)
    def _():
        o_ref[...] = jnp.log(jnp.sum(s_ref[...], axis=1, keepdims=True))


def _lse_pass(xs, waug):
    return pl.pallas_call(
        _lse_body,
        grid=(_NSTEPS,),
        in_specs=[
            pl.BlockSpec((_B, _K), lambda j: (0, 0)),
            pl.BlockSpec((_VB, _K), lambda j: (j, 0)),
        ],
        out_specs=pl.BlockSpec((_B, 1), lambda j: (0, 0)),
        out_shape=jax.ShapeDtypeStruct((_B, 1), jnp.float32),
        scratch_shapes=[
            pltpu.VMEM((_B, 128), jnp.float32),
        ],
        compiler_params=pltpu.CompilerParams(
            dimension_semantics=("arbitrary",),
        ),
    )(xs, waug)


def _out_body(x_ref, w_ref, o_ref):
    o_ref[...] = lax.dot_general(
        x_ref[...], w_ref[...], (((1,), (1,)), ((), ())),
        preferred_element_type=jnp.float32,
    ).astype(jnp.bfloat16)


def _out_pass(xaug, waug):
    return pl.pallas_call(
        _out_body,
        grid=(_NV, _NB),
        in_specs=[
            pl.BlockSpec((_OB, _K), lambda v, b: (b, 0)),
            pl.BlockSpec((_OV, _K), lambda v, b: (v, 0)),
        ],
        out_specs=pl.BlockSpec((_OB, _OV), lambda v, b: (b, v)),
        out_shape=jax.ShapeDtypeStruct((_B, _VOCAB), jnp.bfloat16),
        compiler_params=pltpu.CompilerParams(
            dimension_semantics=("arbitrary", "arbitrary"),
        ),
    )(xaug, waug)


def kernel(word_indices, emb_table, lin_w, lin_b):
    idx2d = word_indices.astype(jnp.int32).reshape(32, -1, 128)
    sum_emb = _make_gather_sum()(idx2d, emb_table)

    w_pad = jnp.pad(lin_w, ((0, _VPAD - _VOCAB), (0, 0)))
    b_pad = jnp.pad(lin_b, (0, _VPAD - _VOCAB), constant_values=-1e30)
    waug = jnp.concatenate(
        [w_pad, b_pad[:, None], jnp.full((_VPAD, 1), -1.0, jnp.float32)],
        axis=1,
    ).astype(jnp.bfloat16)

    ones = jnp.ones((_B, 1), jnp.float32)
    xs = jnp.concatenate(
        [sum_emb * _LOG2E, ones * _LOG2E, ones * 0.0], axis=1
    ).astype(jnp.bfloat16)
    lse = _lse_pass(xs, waug)
    xaug = jnp.concatenate([sum_emb, ones, lse], axis=1).astype(jnp.bfloat16)
    return _out_pass(xaug, waug).astype(jnp.float32)
